# baseline (speedup 1.0000x reference)
import functools

import jax
import jax.numpy as jnp
from jax import lax
from jax.experimental import pallas as pl
from jax.experimental.pallas import tpu as pltpu
from jax.experimental.pallas import tpu_sc as plsc

D0, D1, D2 = 15, 16, 12
EMB = 64
NC, NS, L = 2, 16, 16
NW = NC * NS
G = 128


def _table_body(w0_ref, w1_ref, w2_ref, t_ref):
    w0 = w0_ref[...]
    w1 = w1_ref[...]
    w2 = w2_ref[...]
    t_ref[...] = (w0[:, None, None, :] + w1[None, :, None, :]
                  + w2[None, None, :, :])


def _build_table(w0, w1, w2):
    t4 = pl.pallas_call(
        _table_body,
        out_shape=jax.ShapeDtypeStruct((D0, D1, D2, EMB), jnp.float32),
    )(w0, w1, w2)
    return t4.reshape(D0 * D1 * D2, EMB)


def _make_gather(B, I, J, K):
    n_sb = B * I
    rps = J * K
    sb_per_w = n_sb // NW
    half = sb_per_w // 2
    n_g = rps // G
    n_m = rps // L
    mesh = plsc.VectorSubcoreMesh(core_axis_name="c", subcore_axis_name="s")

    @functools.partial(
        pl.kernel,
        mesh=mesh,
        compiler_params=pltpu.CompilerParams(
            needs_layout_passes=False, use_tc_tiling_on_sc=False),
        out_type=jax.ShapeDtypeStruct((B, I, J, K, EMB), jnp.float32),
        scratch_types=[
            pltpu.VMEM((J, K, 3), jnp.int32),
            pltpu.VMEM((J, K, 3), jnp.int32),
            pltpu.VMEM((n_g, G), jnp.int32),
            pltpu.VMEM((n_g, G), jnp.int32),
            pltpu.VMEM((rps, EMB), jnp.float32),
            pltpu.VMEM((rps, EMB), jnp.float32),
            pltpu.SemaphoreType.DMA,
            pltpu.SemaphoreType.DMA,
            pltpu.SemaphoreType.DMA,
            pltpu.SemaphoreType.DMA,
        ],
    )
    def k(e_hbm, t_hbm, out_hbm, e_v0, e_v1, ci_v0, ci_v1, r_v0, r_v1,
          g_s0, g_s1, s_s0, s_s1):
        wid = lax.axis_index("s") * NC + lax.axis_index("c")
        iota = lax.iota(jnp.int32, L)
        zero = iota * 0

        def fire(sb, e_v, cidx_v, rows_v, gsem):
            b = sb // I
            i = lax.rem(sb, I)
            pltpu.sync_copy(e_hbm.at[b, i], e_v)
            for m in range(n_m):
                r = iota + L * m
                j = r // K
                kk = r - j * K
                e0 = plsc.load_gather(e_v, [j, kk, zero])
                e1 = plsc.load_gather(e_v, [j, kk, zero + 1])
                e2 = plsc.load_gather(e_v, [j, kk, zero + 2])
                c = e0 * (D1 * D2) + e1 * D2 + e2
                cidx_v[(m * L) // G, pl.ds((m * L) % G, L)] = c
            return [
                pltpu.async_copy(
                    t_hbm.at[cidx_v.at[g]],
                    rows_v.at[pl.ds(g * G, G)],
                    gsem,
                )
                for g in range(n_g)
            ]

        def store(sb, rows_v, ssem):
            b = sb // I
            i = lax.rem(sb, I)
            for j in range(J):
                pltpu.async_copy(
                    rows_v.at[pl.ds(j * K, K)], out_hbm.at[b, i, j], ssem)

        def wait_store(ssem):
            for j in range(J):
                pltpu.make_async_copy(
                    r_v0.at[pl.ds(j * K, K)], out_hbm.at[0, 0, j], ssem).wait()

        def body(h, carry):
            sb0 = wid * sb_per_w + 2 * h
            sb1 = sb0 + 1

            @pl.when(h > 0)
            def _():
                wait_store(s_s0)
                wait_store(s_s1)

            cps0 = fire(sb0, e_v0, ci_v0, r_v0, g_s0)
            cps1 = fire(sb1, e_v1, ci_v1, r_v1, g_s1)
            for cp in cps0:
                cp.wait()
            store(sb0, r_v0, s_s0)
            for cp in cps1:
                cp.wait()
            store(sb1, r_v1, s_s1)
            return carry

        lax.fori_loop(0, half, body, 0)
        wait_store(s_s0)
        wait_store(s_s1)

    return k


def kernel(edge_attr, w0, w1, w2):
    B, I, J, K, _ = edge_attr.shape
    t = _build_table(w0, w1, w2)
    return _make_gather(B, I, J, K)(edge_attr.astype(jnp.int32), t)


# speedup vs baseline: 1.3401x; 1.3401x over previous
import functools

import jax
import jax.numpy as jnp
from jax import lax
from jax.experimental import pallas as pl
from jax.experimental.pallas import tpu as pltpu
from jax.experimental.pallas import tpu_sc as plsc

D0, D1, D2 = 15, 16, 12
EMB = 64
NC, NS, L = 2, 16, 16
NW = NC * NS
P = 512
G = 128


def _table_body(w0_ref, w1_ref, w2_ref, t_ref):
    w0 = w0_ref[...]
    w1 = w1_ref[...]
    w2 = w2_ref[...]
    t_ref[...] = (w0[:, None, None, :] + w1[None, :, None, :]
                  + w2[None, None, :, :])


def _build_table(w0, w1, w2):
    t4 = pl.pallas_call(
        _table_body,
        out_shape=jax.ShapeDtypeStruct((D0, D1, D2, EMB), jnp.float32),
    )(w0, w1, w2)
    return t4.reshape(D0 * D1 * D2, EMB)


def _make_gather(n_rows):
    npw = n_rows // NW
    steps = npw // P
    half = steps // 2
    erows = 3 * P // 128
    mesh = plsc.VectorSubcoreMesh(core_axis_name="c", subcore_axis_name="s")

    @functools.partial(
        pl.kernel,
        mesh=mesh,
        compiler_params=pltpu.CompilerParams(
            needs_layout_passes=False, use_tc_tiling_on_sc=False),
        out_type=jax.ShapeDtypeStruct((n_rows, EMB), jnp.float32),
        scratch_types=[
            pltpu.VMEM((erows, 128), jnp.int32),
            pltpu.VMEM((erows, 128), jnp.int32),
            pltpu.VMEM((P // G, G), jnp.int32),
            pltpu.VMEM((P // G, G), jnp.int32),
            pltpu.VMEM((P, EMB), jnp.float32),
            pltpu.VMEM((P, EMB), jnp.float32),
            pltpu.SemaphoreType.DMA,
            pltpu.SemaphoreType.DMA,
            pltpu.SemaphoreType.DMA,
            pltpu.SemaphoreType.DMA,
        ],
    )
    def k(e_hbm, t_hbm, out_hbm, e_v0, e_v1, ci_v0, ci_v1, r_v0, r_v1,
          g_s0, g_s1, s_s0, s_s1):
        wid = lax.axis_index("s") * NC + lax.axis_index("c")
        iota = lax.iota(jnp.int32, L)

        def fire(i, e_v, cidx_v, rows_v, gsem):
            base = wid * npw + i * P
            pltpu.sync_copy(e_hbm.at[pl.ds(3 * base // 128, erows)], e_v)
            for j in range(P // L):
                flat = iota * 3 + (3 * L * j)
                e0 = plsc.load_gather(e_v, [flat // 128, lax.rem(flat, 128)])
                f1 = flat + 1
                e1 = plsc.load_gather(e_v, [f1 // 128, lax.rem(f1, 128)])
                f2 = flat + 2
                e2 = plsc.load_gather(e_v, [f2 // 128, lax.rem(f2, 128)])
                c = e0 * (D1 * D2) + e1 * D2 + e2
                cidx_v[(j * L) // G, pl.ds((j * L) % G, L)] = c
            return [
                pltpu.async_copy(
                    t_hbm.at[cidx_v.at[g]],
                    rows_v.at[pl.ds(g * G, G)],
                    gsem,
                )
                for g in range(P // G)
            ]

        def wait_store(rows_v, ssem):
            pltpu.make_async_copy(rows_v, out_hbm.at[pl.ds(0, P)], ssem).wait()

        def body(h, carry):
            i0, i1 = 2 * h, 2 * h + 1

            @pl.when(h > 0)
            def _():
                wait_store(r_v0, s_s0)
                wait_store(r_v1, s_s1)

            cps0 = fire(i0, e_v0, ci_v0, r_v0, g_s0)
            cps1 = fire(i1, e_v1, ci_v1, r_v1, g_s1)
            for cp in cps0:
                cp.wait()
            pltpu.async_copy(
                r_v0, out_hbm.at[pl.ds(wid * npw + i0 * P, P)], s_s0)
            for cp in cps1:
                cp.wait()
            pltpu.async_copy(
                r_v1, out_hbm.at[pl.ds(wid * npw + i1 * P, P)], s_s1)
            return carry

        lax.fori_loop(0, half, body, 0)
        wait_store(r_v0, s_s0)
        wait_store(r_v1, s_s1)

    return k


def kernel(edge_attr, w0, w1, w2):
    shp = edge_attr.shape
    n_rows = edge_attr.size // 3
    e2d = edge_attr.astype(jnp.int32).reshape(3 * n_rows // 128, 128)
    t = _build_table(w0, w1, w2)
    out = _make_gather(n_rows)(e2d, t)
    return out.reshape(*shp[:-1], EMB)


# speedup vs baseline: 1.3410x; 1.0007x over previous
import functools

import jax
import jax.numpy as jnp
from jax import lax
from jax.experimental import pallas as pl
from jax.experimental.pallas import tpu as pltpu
from jax.experimental.pallas import tpu_sc as plsc

D0, D1, D2 = 15, 16, 12
EMB = 64
NC, NS, L = 2, 16, 16
NW = NC * NS
P = 512
G = 128


def _table_body(w0_ref, w1_ref, w2_ref, t_ref):
    w0 = w0_ref[...]
    w1 = w1_ref[...]
    w2 = w2_ref[...]
    t_ref[...] = (w0[:, None, None, :] + w1[None, :, None, :]
                  + w2[None, None, :, :])


def _build_table(w0, w1, w2):
    t4 = pl.pallas_call(
        _table_body,
        out_shape=jax.ShapeDtypeStruct((D0, D1, D2, EMB), jnp.float32),
    )(w0, w1, w2)
    return t4.reshape(D0 * D1 * D2, EMB)


def _make_gather(n_rows):
    npw = n_rows // NW
    steps = npw // P
    half = steps // 2
    erows = 3 * P // 128
    mesh = plsc.VectorSubcoreMesh(core_axis_name="c", subcore_axis_name="s")

    @functools.partial(
        pl.kernel,
        mesh=mesh,
        compiler_params=pltpu.CompilerParams(
            needs_layout_passes=False, use_tc_tiling_on_sc=False),
        out_type=jax.ShapeDtypeStruct((n_rows, EMB), jnp.float32),
        scratch_types=[
            pltpu.VMEM((erows, 128), jnp.int32),
            pltpu.VMEM((erows, 128), jnp.int32),
            pltpu.VMEM((P // G, G), jnp.int32),
            pltpu.VMEM((P // G, G), jnp.int32),
            pltpu.VMEM((P, EMB), jnp.float32),
            pltpu.VMEM((P, EMB), jnp.float32),
            pltpu.SemaphoreType.DMA,
            pltpu.SemaphoreType.DMA,
            pltpu.SemaphoreType.DMA,
            pltpu.SemaphoreType.DMA,
        ],
    )
    def k(e_hbm, t_hbm, out_hbm, e_v0, e_v1, ci_v0, ci_v1, r_v0, r_v1,
          g_s0, g_s1, s_s0, s_s1):
        wid = lax.axis_index("s") * NC + lax.axis_index("c")
        iota = lax.iota(jnp.int32, L)

        def fire(i, e_v, cidx_v, rows_v, gsem):
            base = wid * npw + i * P
            pltpu.sync_copy(e_hbm.at[pl.ds(3 * base // 128, erows)], e_v)
            for j in range(P // L):
                flat = iota * 3 + (3 * L * j)
                e0 = plsc.load_gather(e_v, [flat // 128, lax.rem(flat, 128)])
                f1 = flat + 1
                e1 = plsc.load_gather(e_v, [f1 // 128, lax.rem(f1, 128)])
                f2 = flat + 2
                e2 = plsc.load_gather(e_v, [f2 // 128, lax.rem(f2, 128)])
                c = e0 * (D1 * D2) + e1 * D2 + e2
                cidx_v[(j * L) // G, pl.ds((j * L) % G, L)] = c
            return [
                pltpu.async_copy(
                    t_hbm.at[cidx_v.at[g]],
                    rows_v.at[pl.ds(g * G, G)],
                    gsem,
                )
                for g in range(P // G)
            ]

        def wait_store(rows_v, ssem):
            pltpu.make_async_copy(rows_v, out_hbm.at[pl.ds(0, P)], ssem).wait()

        def body(h, carry):
            i0, i1 = 2 * h, 2 * h + 1

            @pl.when(h > 0)
            def _():
                wait_store(r_v0, s_s0)
                wait_store(r_v1, s_s1)

            cps0 = fire(i0, e_v0, ci_v0, r_v0, g_s0)
            cps1 = fire(i1, e_v1, ci_v1, r_v1, g_s1)
            for cp in cps0:
                cp.wait()
            pltpu.async_copy(
                r_v0, out_hbm.at[pl.ds(wid * npw + i0 * P, P)], s_s0)
            for cp in cps1:
                cp.wait()
            pltpu.async_copy(
                r_v1, out_hbm.at[pl.ds(wid * npw + i1 * P, P)], s_s1)
            return carry

        lax.fori_loop(0, half, body, 0)
        wait_store(r_v0, s_s0)
        wait_store(r_v1, s_s1)

    return k


def kernel(edge_attr, w0, w1, w2):
    shp = edge_attr.shape
    n_rows = edge_attr.size // 3
    e2d = edge_attr.reshape(3 * n_rows // 128, 128)
    t = _build_table(w0, w1, w2)
    out = _make_gather(n_rows)(e2d, t)
    return out.reshape(*shp[:-1], EMB)
